# Initial kernel scaffold; baseline (speedup 1.0000x reference)
import functools

import jax
import jax.numpy as jnp
from jax import lax
from jax.experimental import pallas as pl
from jax.experimental.pallas import tpu as pltpu
from jax.experimental.pallas import tpu_sc as plsc

RES = (512, 264, 16)
NLVL = 3
NFEAT = 4
NC, NS, L = 2, 16, 16
NW = NC * NS
B = 1024
ROWS_PER_DMA = 128
NDMA = B // ROWS_PER_DMA


def _sc_features(xT, t0, t1, t2):
    n = xT.shape[1]
    pts_per_w = n // NW
    nchunk = pts_per_w // B
    groups = B // L

    mesh = plsc.VectorSubcoreMesh(
        core_axis_name="c", subcore_axis_name="s", num_cores=NC, num_subcores=NS
    )
    scratch = (
        [pltpu.VMEM((B,), jnp.float32),
         pltpu.VMEM((B,), jnp.float32)]
        + [pltpu.VMEM((B,), jnp.int32) for _ in range(NLVL)]
        + [pltpu.VMEM((B, 16), jnp.float32) for _ in range(NLVL)]
        + [pltpu.VMEM((4 * NLVL, B), jnp.float32)]
        + [pltpu.SemaphoreType.DMA for _ in range(NLVL)]
    )

    @functools.partial(
        pl.kernel,
        out_type=jax.ShapeDtypeStruct((4 * NLVL, n), jnp.float32),
        mesh=mesh,
        scratch_types=scratch,
    )
    def k(xT_hbm, t0_hbm, t1_hbm, t2_hbm, out_hbm,
          u_ref, v_ref, i0, i1, i2, r0, r1, r2, f_ref, s0, s1, s2):
        t_hbm = (t0_hbm, t1_hbm, t2_hbm)
        idx_refs = (i0, i1, i2)
        row_refs = (r0, r1, r2)
        sems = (s0, s1, s2)
        wid = lax.axis_index("s") * NC + lax.axis_index("c")
        iota = lax.iota(jnp.int32, L)

        def chunk_body(ci, carry):
            base = wid * pts_per_w + ci * B
            pltpu.sync_copy(xT_hbm.at[1, pl.ds(base, B)], u_ref)
            pltpu.sync_copy(xT_hbm.at[2, pl.ds(base, B)], v_ref)

            def idx_body(j, c):
                sl = pl.ds(j * L, L)
                u = u_ref[sl]
                v = v_ref[sl]
                for lvl, r in enumerate(RES):
                    x0 = (u * r).astype(jnp.int32)
                    y0 = (v * r).astype(jnp.int32)
                    idx_refs[lvl][sl] = y0 * r + x0
                return c

            lax.fori_loop(0, groups, idx_body, 0)

            dmas = []
            for lvl in range(NLVL):
                for g in range(NDMA):
                    dmas.append(pltpu.async_copy(
                        t_hbm[lvl].at[idx_refs[lvl].at[pl.ds(g * ROWS_PER_DMA, ROWS_PER_DMA)]],
                        row_refs[lvl].at[pl.ds(g * ROWS_PER_DMA, ROWS_PER_DMA)],
                        sems[lvl],
                    ))

            for lvl, r in enumerate(RES):
                for g in range(NDMA):
                    dmas[lvl * NDMA + g].wait()

                def interp_body(j, c, lvl=lvl, r=r):
                    sl = pl.ds(j * L, L)
                    u = u_ref[sl]
                    v = v_ref[sl]
                    fx = u * r
                    x0 = fx.astype(jnp.int32)
                    wx = fx - x0.astype(jnp.float32)
                    fy = v * r
                    y0 = fy.astype(jnp.int32)
                    wy = fy - y0.astype(jnp.float32)
                    ridx = iota + j * L
                    vals = [
                        plsc.load_gather(
                            row_refs[lvl],
                            [ridx, jnp.full((L,), cf, dtype=jnp.int32)],
                        )
                        for cf in range(16)
                    ]
                    for f in range(NFEAT):
                        v00 = vals[0 + f]
                        v10 = vals[4 + f]
                        v01 = vals[8 + f]
                        v11 = vals[12 + f]
                        fup = v00 + wx * (v10 - v00)
                        fdn = v01 + wx * (v11 - v01)
                        f_ref[lvl * NFEAT + f, sl] = fup + wy * (fdn - fup)
                    return c

                lax.fori_loop(0, groups, interp_body, 0)

            for fi in range(4 * NLVL):
                pltpu.sync_copy(f_ref.at[fi], out_hbm.at[fi, pl.ds(base, B)])
            return carry

        lax.fori_loop(0, nchunk, chunk_body, 0)

    return k(xT, t0, t1, t2)


def _mlp_body(f_ref, idf_ref, w0_ref, b0_ref, w1_ref, b1_ref, w2_ref, b2_ref, o_ref):
    h = jnp.concatenate([idf_ref[...], f_ref[...]], axis=0)
    h = jnp.maximum(
        jnp.dot(w0_ref[...], h, preferred_element_type=jnp.float32) + b0_ref[...], 0.0)
    h = jnp.maximum(
        jnp.dot(w1_ref[...], h, preferred_element_type=jnp.float32) + b1_ref[...], 0.0)
    o_ref[...] = jnp.dot(w2_ref[...], h, preferred_element_type=jnp.float32) + b2_ref[...]


def _mlp(feats, idf, W0, b0, W1, b1, W2, b2):
    n = feats.shape[1]
    nb = 2048
    full = lambda i: (0, 0)
    return pl.pallas_call(
        _mlp_body,
        grid=(n // nb,),
        in_specs=[
            pl.BlockSpec((4 * NLVL, nb), lambda i: (0, i)),
            pl.BlockSpec((1, nb), lambda i: (0, i)),
            pl.BlockSpec(W0.shape, full),
            pl.BlockSpec((W0.shape[0], 1), full),
            pl.BlockSpec(W1.shape, full),
            pl.BlockSpec((W1.shape[0], 1), full),
            pl.BlockSpec(W2.shape, full),
            pl.BlockSpec((W2.shape[0], 1), full),
        ],
        out_specs=pl.BlockSpec((W2.shape[0], nb), lambda i: (0, i)),
        out_shape=jax.ShapeDtypeStruct((W2.shape[0], n), jnp.float32),
    )(feats, idf, W0, b0, W1, b1, W2, b2)


def kernel(x, emb0, emb1, emb2, W0, b0, W1, b1, W2, b2):
    xT = x.T
    tables = []
    for r, emb in zip(RES, (emb0, emb1, emb2)):
        s = (r + 1) * (r + 1)
        tables.append(jnp.concatenate(
            [emb[: s - r - 1], emb[1 : s - r], emb[r : s - 1], emb[r + 1 : s]],
            axis=1,
        ))
    feats = _sc_features(xT, *tables)
    out = _mlp(feats, xT[0:1], W0, b0[:, None], W1, b1[:, None], W2, b2[:, None])
    return out.T


# speedup vs baseline: 8.7122x; 8.7122x over previous
import functools

import jax
import jax.numpy as jnp
from jax import lax
from jax.experimental import pallas as pl
from jax.experimental.pallas import tpu as pltpu
from jax.experimental.pallas import tpu_sc as plsc

RES = (512, 264, 16)
NLVL = 3
NC, NS, L = 2, 16, 16
NW = NC * NS
B = 1024
ROWS_PER_DMA = 128
NDMA = B // ROWS_PER_DMA


def _sc_gather(xflat, n, t0, t1, t2):
    pts_per_w = n // NW
    nchunk = pts_per_w // B
    groups = B // L

    mesh = plsc.VectorSubcoreMesh(
        core_axis_name="c", subcore_axis_name="s", num_cores=NC, num_subcores=NS
    )
    scratch = (
        [pltpu.VMEM((B,), jnp.float32),
         pltpu.VMEM((B,), jnp.float32)]
        + [pltpu.VMEM((B,), jnp.int32) for _ in range(NLVL)]
        + [pltpu.VMEM((B, 16), jnp.float32) for _ in range(NLVL)]
        + [pltpu.SemaphoreType.DMA for _ in range(NLVL)]
    )

    @functools.partial(
        pl.kernel,
        out_type=tuple(jax.ShapeDtypeStruct((n, 16), jnp.float32)
                       for _ in range(NLVL)),
        mesh=mesh,
        scratch_types=scratch,
        compiler_params=pltpu.CompilerParams(use_tc_tiling_on_sc=False),
    )
    def k(x_hbm, t0_hbm, t1_hbm, t2_hbm, o0, o1, o2,
          u_ref, v_ref, i0, i1, i2, r0, r1, r2, s0, s1, s2):
        t_hbm = (t0_hbm, t1_hbm, t2_hbm)
        out_hbm = (o0, o1, o2)
        idx_refs = (i0, i1, i2)
        row_refs = (r0, r1, r2)
        sems = (s0, s1, s2)
        wid = lax.axis_index("s") * NC + lax.axis_index("c")

        def chunk_body(ci, carry):
            base = wid * pts_per_w + ci * B
            pltpu.sync_copy(x_hbm.at[pl.ds(n + base, B)], u_ref)
            pltpu.sync_copy(x_hbm.at[pl.ds(2 * n + base, B)], v_ref)

            def idx_body(j, c):
                sl = pl.ds(j * L, L)
                u = u_ref[sl]
                v = v_ref[sl]
                for lvl, r in enumerate(RES):
                    x0 = (u * r).astype(jnp.int32)
                    y0 = (v * r).astype(jnp.int32)
                    idx_refs[lvl][sl] = y0 * r + x0
                return c

            lax.fori_loop(0, groups, idx_body, 0)

            dmas = []
            for lvl in range(NLVL):
                for g in range(NDMA):
                    dmas.append(pltpu.async_copy(
                        t_hbm[lvl].at[idx_refs[lvl].at[pl.ds(g * ROWS_PER_DMA, ROWS_PER_DMA)]],
                        row_refs[lvl].at[pl.ds(g * ROWS_PER_DMA, ROWS_PER_DMA)],
                        sems[lvl],
                    ))
            for lvl in range(NLVL):
                for g in range(NDMA):
                    dmas[lvl * NDMA + g].wait()
                pltpu.sync_copy(row_refs[lvl], out_hbm[lvl].at[pl.ds(base, B), :])
            return carry

        lax.fori_loop(0, nchunk, chunk_body, 0)

    return k(xflat, t0, t1, t2)


def _tc_body(c0, c1, c2, u8, v8, i8, eb, g0, g1, g2, gi, b0t, w1t, b1r, w2t, b2r,
             *out_refs):
    e = eb[...]
    u16 = jnp.dot(u8[...], e, preferred_element_type=jnp.float32)
    v16 = jnp.dot(v8[...], e, preferred_element_type=jnp.float32)
    cf = jax.lax.broadcasted_iota(jnp.int32, (1, 128), 1) % 16
    is_x1 = (cf // 4) % 2 == 1
    is_y1 = (cf // 8) == 1
    acc = jnp.dot(i8[...], gi[...], preferred_element_type=jnp.float32)
    corners = (c0, c1, c2)
    gs = (g0, g1, g2)
    for lvl, r in enumerate(RES):
        fx = u16 * r
        wx = fx - jnp.floor(fx)
        fy = v16 * r
        wy = fy - jnp.floor(fy)
        wxs = jnp.where(is_x1, wx, 1.0 - wx)
        wys = jnp.where(is_y1, wy, 1.0 - wy)
        t = corners[lvl][...] * wxs * wys
        acc = acc + jnp.dot(t, gs[lvl][...], preferred_element_type=jnp.float32)
    h1 = jnp.maximum(acc + b0t[...], 0.0)
    for j in range(8):
        hj = h1[:, 64 * j:64 * j + 64]
        h2 = jnp.maximum(
            jnp.dot(hj, w1t[...], preferred_element_type=jnp.float32) + b1r[...], 0.0)
        out_refs[j][...] = (
            jnp.dot(h2, w2t[...], preferred_element_type=jnp.float32) + b2r[...])


def _tc_mlp(c0, c1, c2, u8, v8, i8, W0, b0, W1, b1, W2, b2):
    m = u8.shape[0]
    mb = 512
    grid = (m // mb,)
    eye8 = jnp.eye(8, dtype=jnp.float32)
    eb = jnp.kron(eye8, jnp.ones((1, 16), jnp.float32))
    gi = jnp.kron(eye8, W0[:, 0:1].T)
    gs = []
    for lvl in range(NLVL):
        e16 = jnp.tile(W0[:, 1 + 4 * lvl:5 + 4 * lvl].T, (4, 1))
        gs.append(jnp.kron(eye8, e16))
    b0t = jnp.tile(b0, 8)[None, :]
    row = lambda i: (i, 0)
    full = lambda i: (0, 0)
    out_dim = W2.shape[0]
    outs = pl.pallas_call(
        _tc_body,
        grid=grid,
        in_specs=[
            pl.BlockSpec((mb, 128), row),
            pl.BlockSpec((mb, 128), row),
            pl.BlockSpec((mb, 128), row),
            pl.BlockSpec((mb, 8), row),
            pl.BlockSpec((mb, 8), row),
            pl.BlockSpec((mb, 8), row),
            pl.BlockSpec((8, 128), full),
            pl.BlockSpec((128, 512), full),
            pl.BlockSpec((128, 512), full),
            pl.BlockSpec((128, 512), full),
            pl.BlockSpec((8, 512), full),
            pl.BlockSpec((1, 512), full),
            pl.BlockSpec((64, 64), full),
            pl.BlockSpec((1, 64), full),
            pl.BlockSpec((64, out_dim), full),
            pl.BlockSpec((1, out_dim), full),
        ],
        out_specs=[pl.BlockSpec((mb, out_dim), row) for _ in range(8)],
        out_shape=[jax.ShapeDtypeStruct((m, out_dim), jnp.float32)
                   for _ in range(8)],
    )(c0, c1, c2, u8, v8, i8, eb, gs[0], gs[1], gs[2], gi, b0t,
      W1.T, b1[None, :], W2.T, b2[None, :])
    return outs


def kernel(x, emb0, emb1, emb2, W0, b0, W1, b1, W2, b2):
    n = x.shape[0]
    xT = x.T
    tables = []
    for r, emb in zip(RES, (emb0, emb1, emb2)):
        s = (r + 1) * (r + 1)
        tables.append(jnp.concatenate(
            [emb[: s - r - 1], emb[1 : s - r], emb[r : s - 1], emb[r + 1 : s]],
            axis=1,
        ))
    c0, c1, c2 = _sc_gather(xT.reshape(-1), n, *tables)
    m = n // 8
    packed = lambda c: c.reshape(m, 128)
    u8 = xT[1].reshape(m, 8)
    v8 = xT[2].reshape(m, 8)
    i8 = xT[0].reshape(m, 8)
    outs = _tc_mlp(packed(c0), packed(c1), packed(c2), u8, v8, i8,
                   W0, b0, W1, b1, W2, b2)
    return jnp.stack(outs, axis=1).reshape(n, W2.shape[0])
